# Initial kernel scaffold; baseline (speedup 1.0000x reference)
import jax
import jax.numpy as jnp
from jax.experimental import pallas as pl


def kernel(xyz, feat):
    raise NotImplementedError("write your pallas kernel here")


# speedup vs baseline: 10.6829x; 10.6829x over previous
import functools

import jax
import jax.numpy as jnp
from jax import lax
from jax.experimental import pallas as pl
from jax.experimental.pallas import tpu as pltpu
from jax.experimental.pallas import tpu_sc as plsc

D = 48
F = 19
N = 20000
NPAD = 20480
CH = 2048
NCHUNK = NPAD // CH
GPC = CH // 16
NVOX = D * D * D
SENT = 60


def _bin_body(xyz_ref, feat_ref, cc_ref, w_ref):
    x = xyz_ref[0:1, :]
    y = xyz_ref[1:2, :]
    z = xyz_ref[2:3, :]
    m = (x != 0.0) & (y != 0.0) & (z != 0.0)

    def bin1(v):
        mn = jnp.min(jnp.where(m, v, jnp.inf), axis=1, keepdims=True)
        mx = jnp.max(jnp.where(m, v, -jnp.inf), axis=1, keepdims=True)
        c = jnp.floor((v - mn) / (mx - mn) * (D - 1)).astype(jnp.int32)
        return jnp.where(m, jnp.clip(c, 0, D - 1), SENT)

    cc_ref[0:1, :] = bin1(z)
    cc_ref[1:2, :] = bin1(y)
    cc_ref[2:3, :] = bin1(x)
    w_ref[...] = feat_ref[...] * m.astype(jnp.float32)


def _axis_ok(c, d):
    if d < 0:
        return (c >= -d) & (c <= D - 1 - d)
    return c <= D - 1 - d


def _sc_body(cz_hbm, cy_hbm, cx_hbm, w_hbm, out_hbm, acc, czv, cyv, cxv, wv):
    fid = lax.axis_index("s") * 2 + lax.axis_index("c")

    @pl.when(fid < F)
    def _():
        zeros16 = jnp.zeros((16,), jnp.float32)

        def zero_body(i, _):
            acc[pl.ds(i * 16, 16)] = zeros16
            return 0

        lax.fori_loop(0, NVOX // 16, zero_body, 0)

        def chunk_body(ci, _):
            base = ci * CH
            pltpu.sync_copy(cz_hbm.at[pl.ds(base, CH)], czv)
            pltpu.sync_copy(cy_hbm.at[pl.ds(base, CH)], cyv)
            pltpu.sync_copy(cx_hbm.at[pl.ds(base, CH)], cxv)
            pltpu.sync_copy(w_hbm.at[fid, pl.ds(base, CH)], wv)

            def group_body(g, _):
                off = pl.multiple_of(g * 16, 16)
                cz = czv[pl.ds(off, 16)]
                cy = cyv[pl.ds(off, 16)]
                cx = cxv[pl.ds(off, 16)]
                w = wv[pl.ds(off, 16)]
                vbase = cz * (D * D) + cy * D + cx
                okz = {dz: _axis_ok(cz, dz) for dz in (-1, 0, 1)}
                oky = {dy: _axis_ok(cy, dy) for dy in (-1, 0, 1)}
                okx = {dx: _axis_ok(cx, dx) for dx in (-1, 0, 1)}
                for dz in (-1, 0, 1):
                    for dy in (-1, 0, 1):
                        mzy = okz[dz] & oky[dy]
                        for dx in (-1, 0, 1):
                            k = dz * (D * D) + dy * D + dx
                            plsc.addupdate_scatter(
                                acc, [vbase + k], w, mask=mzy & okx[dx]
                            )
                return 0

            lax.fori_loop(0, GPC, group_body, 0)
            return 0

        lax.fori_loop(0, NCHUNK, chunk_body, 0)
        pltpu.sync_copy(acc, out_hbm.at[fid])


@jax.jit
def kernel(xyz, feat):
    xyzT = jnp.zeros((3, NPAD), jnp.float32).at[:, :N].set(xyz.T)
    featT = jnp.zeros((F, NPAD), jnp.float32).at[:, :N].set(feat.T)

    cc, wT = pl.pallas_call(
        _bin_body,
        out_shape=[
            jax.ShapeDtypeStruct((3, NPAD), jnp.int32),
            jax.ShapeDtypeStruct((F, NPAD), jnp.float32),
        ],
    )(xyzT, featT)

    sc = pl.kernel(
        _sc_body,
        out_type=jax.ShapeDtypeStruct((F, NVOX), jnp.float32),
        mesh=plsc.VectorSubcoreMesh(core_axis_name="c", subcore_axis_name="s"),
        compiler_params=pltpu.CompilerParams(needs_layout_passes=False),
        scratch_types=[
            pltpu.VMEM((NVOX,), jnp.float32),
            pltpu.VMEM((CH,), jnp.int32),
            pltpu.VMEM((CH,), jnp.int32),
            pltpu.VMEM((CH,), jnp.int32),
            pltpu.VMEM((CH,), jnp.float32),
        ],
    )
    flat = sc(cc[0], cc[1], cc[2], wT)
    return flat.reshape(F, D, D, D)


# speedup vs baseline: 13.6465x; 1.2774x over previous
import functools

import jax
import jax.numpy as jnp
from jax import lax
from jax.experimental import pallas as pl
from jax.experimental.pallas import tpu as pltpu
from jax.experimental.pallas import tpu_sc as plsc

D = 48
F = 19
N = 20000
NPAD = 20480
CH = 1024
NCHUNK = NPAD // CH
GPC = CH // 16
NVOX = D * D * D
SENT = 60


def _bin_body(xyz_ref, feat_ref, cc_ref, w_ref):
    x = xyz_ref[0:1, :]
    y = xyz_ref[1:2, :]
    z = xyz_ref[2:3, :]
    m = (x != 0.0) & (y != 0.0) & (z != 0.0)

    def bin1(v):
        mn = jnp.min(jnp.where(m, v, jnp.inf), axis=1, keepdims=True)
        mx = jnp.max(jnp.where(m, v, -jnp.inf), axis=1, keepdims=True)
        c = jnp.floor((v - mn) / (mx - mn) * (D - 1)).astype(jnp.int32)
        return jnp.where(m, jnp.clip(c, 0, D - 1), SENT)

    cc_ref[0:1, :] = bin1(z)
    cc_ref[1:2, :] = bin1(y)
    cc_ref[2:3, :] = bin1(x)
    w_ref[...] = feat_ref[...] * m.astype(jnp.float32)


def _axis_ok(c, d):
    if d < 0:
        return (c >= -d) & (c <= D - 1 - d)
    return c <= D - 1 - d


def _sc_body(
    cz_hbm, cy_hbm, cx_hbm, w_hbm, out_hbm,
    acc, cz0, cy0, cx0, w0, cz1, cy1, cx1, w1, sem0, sem1,
):
    fid = lax.axis_index("s") * 2 + lax.axis_index("c")
    bufsets = ((cz0, cy0, cx0, w0), (cz1, cy1, cx1, w1))
    sems = (sem0, sem1)

    @pl.when(fid < F)
    def _():
        zeros16 = jnp.zeros((16,), jnp.float32)

        def fire(ci):
            base = ci * CH
            bz, by, bx, bw = bufsets[ci & 1]
            sem = sems[ci & 1]
            return [
                pltpu.async_copy(cz_hbm.at[pl.ds(base, CH)], bz, sem),
                pltpu.async_copy(cy_hbm.at[pl.ds(base, CH)], by, sem),
                pltpu.async_copy(cx_hbm.at[pl.ds(base, CH)], bx, sem),
                pltpu.async_copy(w_hbm.at[fid, pl.ds(base, CH)], bw, sem),
            ]

        hs = fire(0)

        @plsc.parallel_loop(0, NVOX, step=16, unroll=8)
        def _zero(i):
            acc[pl.ds(i, 16)] = zeros16

        for ci in range(NCHUNK):
            hs_next = fire(ci + 1) if ci + 1 < NCHUNK else None
            for h in hs:
                h.wait()
            hs = hs_next
            bz, by, bx, bw = bufsets[ci & 1]

            @plsc.parallel_loop(0, CH, step=16, unroll=2)
            def _group(off):
                cz = bz[pl.ds(off, 16)]
                cy = by[pl.ds(off, 16)]
                cx = bx[pl.ds(off, 16)]
                w = bw[pl.ds(off, 16)]
                vbase = cz * (D * D) + cy * D + cx
                okz = {dz: _axis_ok(cz, dz) for dz in (-1, 0, 1)}
                oky = {dy: _axis_ok(cy, dy) for dy in (-1, 0, 1)}
                okx = {dx: _axis_ok(cx, dx) for dx in (-1, 0, 1)}
                for dz in (-1, 0, 1):
                    for dy in (-1, 0, 1):
                        mzy = okz[dz] & oky[dy]
                        for dx in (-1, 0, 1):
                            k = dz * (D * D) + dy * D + dx
                            plsc.addupdate_scatter(
                                acc, [vbase + k], w, mask=mzy & okx[dx]
                            )

        pltpu.sync_copy(acc, out_hbm.at[fid])


@jax.jit
def kernel(xyz, feat):
    xyzT = jnp.zeros((3, NPAD), jnp.float32).at[:, :N].set(xyz.T)
    featT = jnp.zeros((F, NPAD), jnp.float32).at[:, :N].set(feat.T)

    cc, wT = pl.pallas_call(
        _bin_body,
        out_shape=[
            jax.ShapeDtypeStruct((3, NPAD), jnp.int32),
            jax.ShapeDtypeStruct((F, NPAD), jnp.float32),
        ],
    )(xyzT, featT)

    sc = pl.kernel(
        _sc_body,
        out_type=jax.ShapeDtypeStruct((F, NVOX), jnp.float32),
        mesh=plsc.VectorSubcoreMesh(core_axis_name="c", subcore_axis_name="s"),
        compiler_params=pltpu.CompilerParams(needs_layout_passes=False),
        scratch_types=[
            pltpu.VMEM((NVOX,), jnp.float32),
            pltpu.VMEM((CH,), jnp.int32),
            pltpu.VMEM((CH,), jnp.int32),
            pltpu.VMEM((CH,), jnp.int32),
            pltpu.VMEM((CH,), jnp.float32),
            pltpu.VMEM((CH,), jnp.int32),
            pltpu.VMEM((CH,), jnp.int32),
            pltpu.VMEM((CH,), jnp.int32),
            pltpu.VMEM((CH,), jnp.float32),
            pltpu.SemaphoreType.DMA,
            pltpu.SemaphoreType.DMA,
        ],
    )
    flat = sc(cc[0], cc[1], cc[2], wT)
    return flat.reshape(F, D, D, D)


# speedup vs baseline: 22.6628x; 1.6607x over previous
import functools

import jax
import jax.numpy as jnp
from jax import lax
from jax.experimental import pallas as pl
from jax.experimental.pallas import tpu as pltpu
from jax.experimental.pallas import tpu_sc as plsc

D = 48
F = 19
N = 20000
NPAD = 20480
CH = 2048
NCHUNK = NPAD // CH
NVOX = D * D * D


def _bin_body(xyz_ref, feat_ref, vb_ref, w_ref):
    x = xyz_ref[0:1, :]
    y = xyz_ref[1:2, :]
    z = xyz_ref[2:3, :]
    m = (x != 0.0) & (y != 0.0) & (z != 0.0)

    def bin1(v):
        mn = jnp.min(jnp.where(m, v, jnp.inf), axis=1, keepdims=True)
        mx = jnp.max(jnp.where(m, v, -jnp.inf), axis=1, keepdims=True)
        c = jnp.floor((v - mn) / (mx - mn) * (D - 1)).astype(jnp.int32)
        return jnp.clip(c, 0, D - 1)

    vb = bin1(z) * (D * D) + bin1(y) * D + bin1(x)
    vb_ref[...] = jnp.where(m, vb, NVOX)
    w_ref[...] = feat_ref[...] * m.astype(jnp.float32)


def _sc_body(vb_hbm, w_hbm, out_hbm, acc, vb0, w0, vb1, w1, sem0, sem1):
    fid = lax.axis_index("s") * 2 + lax.axis_index("c")
    bufsets = ((vb0, w0), (vb1, w1))
    sems = (sem0, sem1)

    @pl.when(fid < F)
    def _():
        zeros16 = jnp.zeros((16,), jnp.float32)

        def fire(ci):
            base = ci * CH
            bv, bw = bufsets[ci & 1]
            sem = sems[ci & 1]
            return [
                pltpu.async_copy(vb_hbm.at[pl.ds(base, CH)], bv, sem),
                pltpu.async_copy(w_hbm.at[fid, pl.ds(base, CH)], bw, sem),
            ]

        hs = fire(0)

        @plsc.parallel_loop(0, NVOX, step=16, unroll=8)
        def _zero(i):
            acc[pl.ds(i, 16)] = zeros16

        for ci in range(NCHUNK):
            hs_next = fire(ci + 1) if ci + 1 < NCHUNK else None
            for h in hs:
                h.wait()
            hs = hs_next
            bv, bw = bufsets[ci & 1]

            @plsc.parallel_loop(0, CH, step=16, unroll=4)
            def _group(off):
                vb = bv[pl.ds(off, 16)]
                w = bw[pl.ds(off, 16)]
                plsc.addupdate_scatter(acc, [vb], w, mask=vb < NVOX)

        pltpu.sync_copy(acc, out_hbm.at[fid])


def _conv_body(a_ref, o_ref):
    a = a_ref[...]
    col = lax.broadcasted_iota(jnp.int32, (D, D * D), 1)
    xm = col % D
    z1 = jnp.zeros((D, 1), jnp.float32)
    xl = jnp.concatenate([a[:, 1:], z1], axis=1)
    xr = jnp.concatenate([z1, a[:, : D * D - 1]], axis=1)
    t = a + jnp.where(xm == D - 1, 0.0, xl) + jnp.where(xm == 0, 0.0, xr)
    zd = jnp.zeros((D, D), jnp.float32)
    t = (
        t
        + jnp.concatenate([t[:, D:], zd], axis=1)
        + jnp.concatenate([zd, t[:, : D * D - D]], axis=1)
    )
    zr = jnp.zeros((1, D * D), jnp.float32)
    o_ref[...] = (
        t
        + jnp.concatenate([t[1:, :], zr], axis=0)
        + jnp.concatenate([zr, t[: D - 1, :]], axis=0)
    )


@jax.jit
def kernel(xyz, feat):
    xyzT = jnp.zeros((3, NPAD), jnp.float32).at[:, :N].set(xyz.T)
    featT = jnp.zeros((F, NPAD), jnp.float32).at[:, :N].set(feat.T)

    vb2, wT = pl.pallas_call(
        _bin_body,
        out_shape=[
            jax.ShapeDtypeStruct((1, NPAD), jnp.int32),
            jax.ShapeDtypeStruct((F, NPAD), jnp.float32),
        ],
    )(xyzT, featT)

    sc = pl.kernel(
        _sc_body,
        out_type=jax.ShapeDtypeStruct((F, NVOX), jnp.float32),
        mesh=plsc.VectorSubcoreMesh(core_axis_name="c", subcore_axis_name="s"),
        compiler_params=pltpu.CompilerParams(needs_layout_passes=False),
        scratch_types=[
            pltpu.VMEM((NVOX,), jnp.float32),
            pltpu.VMEM((CH,), jnp.int32),
            pltpu.VMEM((CH,), jnp.float32),
            pltpu.VMEM((CH,), jnp.int32),
            pltpu.VMEM((CH,), jnp.float32),
            pltpu.SemaphoreType.DMA,
            pltpu.SemaphoreType.DMA,
        ],
    )
    planes = sc(vb2.reshape(NPAD), wT)

    out = pl.pallas_call(
        _conv_body,
        grid=(F,),
        in_specs=[pl.BlockSpec((None, D, D * D), lambda i: (i, 0, 0))],
        out_specs=pl.BlockSpec((None, D, D * D), lambda i: (i, 0, 0)),
        out_shape=jax.ShapeDtypeStruct((F, D, D * D), jnp.float32),
    )(planes.reshape(F, D, D * D))
    return out.reshape(F, D, D, D)


# speedup vs baseline: 25.5005x; 1.1252x over previous
import functools

import jax
import jax.numpy as jnp
from jax import lax
from jax.experimental import pallas as pl
from jax.experimental.pallas import tpu as pltpu
from jax.experimental.pallas import tpu_sc as plsc

D = 48
F = 19
N = 20000
NPAD = 20480
CH = 2048
NCHUNK = NPAD // CH
NVOX = D * D * D


def _bin_body(xyz_ref, feat_ref, vb_ref, w_ref):
    x = xyz_ref[0:1, :]
    y = xyz_ref[1:2, :]
    z = xyz_ref[2:3, :]
    m = (x != 0.0) & (y != 0.0) & (z != 0.0)

    def bin1(v):
        mn = jnp.min(jnp.where(m, v, jnp.inf), axis=1, keepdims=True)
        mx = jnp.max(jnp.where(m, v, -jnp.inf), axis=1, keepdims=True)
        c = jnp.floor((v - mn) / (mx - mn) * (D - 1)).astype(jnp.int32)
        return jnp.clip(c, 0, D - 1)

    vb = bin1(z) * (D * D) + bin1(y) * D + bin1(x)
    vb_ref[...] = jnp.where(m, vb, NVOX)
    w_ref[...] = feat_ref[...] * m.astype(jnp.float32)


def _sc_body(vb_hbm, w_hbm, out_hbm, acc, vb0, w0, vb1, w1, sem0, sem1):
    fid = lax.axis_index("s") * 2 + lax.axis_index("c")
    bufsets = ((vb0, w0), (vb1, w1))
    sems = (sem0, sem1)

    @pl.when(fid < F)
    def _():
        zeros16 = jnp.zeros((16,), jnp.float32)

        def fire(ci):
            base = ci * CH
            bv, bw = bufsets[ci & 1]
            sem = sems[ci & 1]
            return [
                pltpu.async_copy(vb_hbm.at[pl.ds(base, CH)], bv, sem),
                pltpu.async_copy(w_hbm.at[fid, pl.ds(base, CH)], bw, sem),
            ]

        hs = fire(0)

        @plsc.parallel_loop(0, NVOX, step=16, unroll=8)
        def _zero(i):
            acc[pl.ds(i, 16)] = zeros16

        for ci in range(NCHUNK):
            hs_next = fire(ci + 1) if ci + 1 < NCHUNK else None
            for h in hs:
                h.wait()
            hs = hs_next
            bv, bw = bufsets[ci & 1]

            @plsc.parallel_loop(0, CH, step=16, unroll=4)
            def _group(off):
                vb = bv[pl.ds(off, 16)]
                w = bw[pl.ds(off, 16)]
                plsc.addupdate_scatter(acc, [vb], w, mask=vb < NVOX)

        pltpu.sync_copy(acc, out_hbm.at[fid])


def _conv_body(p_hbm, o_ref, buf, sem):
    f = pl.program_id(0)
    par = f % 2
    nxt = (f + 1) % 2

    @pl.when(f == 0)
    def _():
        pltpu.make_async_copy(p_hbm.at[0], buf.at[0], sem.at[0]).start()

    @pl.when(f + 1 < F)
    def _():
        pltpu.make_async_copy(p_hbm.at[f + 1], buf.at[nxt], sem.at[nxt]).start()

    pltpu.make_async_copy(p_hbm.at[f], buf.at[par], sem.at[par]).wait()

    zrow = jnp.zeros((1, D), jnp.float32)
    zcol = jnp.zeros((D, 1), jnp.float32)
    for z in range(D):
        zs = buf[par, z]
        if z > 0:
            zs = zs + buf[par, z - 1]
        if z < D - 1:
            zs = zs + buf[par, z + 1]
        t = (
            zs
            + jnp.concatenate([zs[1:, :], zrow], axis=0)
            + jnp.concatenate([zrow, zs[: D - 1, :]], axis=0)
        )
        o_ref[0, z] = (
            t
            + jnp.concatenate([t[:, 1:], zcol], axis=1)
            + jnp.concatenate([zcol, t[:, : D - 1]], axis=1)
        )


@jax.jit
def kernel(xyz, feat):
    xyzT = jnp.zeros((3, NPAD), jnp.float32).at[:, :N].set(xyz.T)
    featT = jnp.zeros((F, NPAD), jnp.float32).at[:, :N].set(feat.T)

    vb2, wT = pl.pallas_call(
        _bin_body,
        out_shape=[
            jax.ShapeDtypeStruct((1, NPAD), jnp.int32),
            jax.ShapeDtypeStruct((F, NPAD), jnp.float32),
        ],
    )(xyzT, featT)

    sc = pl.kernel(
        _sc_body,
        out_type=jax.ShapeDtypeStruct((F, NVOX), jnp.float32),
        mesh=plsc.VectorSubcoreMesh(core_axis_name="c", subcore_axis_name="s"),
        compiler_params=pltpu.CompilerParams(needs_layout_passes=False),
        scratch_types=[
            pltpu.VMEM((NVOX,), jnp.float32),
            pltpu.VMEM((CH,), jnp.int32),
            pltpu.VMEM((CH,), jnp.float32),
            pltpu.VMEM((CH,), jnp.int32),
            pltpu.VMEM((CH,), jnp.float32),
            pltpu.SemaphoreType.DMA,
            pltpu.SemaphoreType.DMA,
        ],
    )
    planes = sc(vb2.reshape(NPAD), wT)

    return pl.pallas_call(
        _conv_body,
        grid=(F,),
        in_specs=[pl.BlockSpec(memory_space=pl.ANY)],
        out_specs=pl.BlockSpec((1, D, D, D), lambda i: (i, 0, 0, 0)),
        out_shape=jax.ShapeDtypeStruct((F, D, D, D), jnp.float32),
        scratch_shapes=[
            pltpu.VMEM((2, D, D, D), jnp.float32),
            pltpu.SemaphoreType.DMA((2,)),
        ],
    )(planes.reshape(F, D, D, D))


# speedup vs baseline: 25.5589x; 1.0023x over previous
import functools

import jax
import jax.numpy as jnp
from jax import lax
from jax.experimental import pallas as pl
from jax.experimental.pallas import tpu as pltpu
from jax.experimental.pallas import tpu_sc as plsc

D = 48
F = 19
N = 20000
NPAD = 20480
CH = 2048
NCHUNK = NPAD // CH
NVOX = D * D * D


def _bin_body(xyz_ref, feat_ref, vb_ref, w_ref):
    x = xyz_ref[0:1, :]
    y = xyz_ref[1:2, :]
    z = xyz_ref[2:3, :]
    m = (x != 0.0) & (y != 0.0) & (z != 0.0)

    def bin1(v):
        mn = jnp.min(jnp.where(m, v, jnp.inf), axis=1, keepdims=True)
        mx = jnp.max(jnp.where(m, v, -jnp.inf), axis=1, keepdims=True)
        c = jnp.floor((v - mn) / (mx - mn) * (D - 1)).astype(jnp.int32)
        return jnp.clip(c, 0, D - 1)

    vb = bin1(z) * (D * D) + bin1(y) * D + bin1(x)
    vb_ref[...] = jnp.where(m, vb, NVOX)
    w_ref[...] = feat_ref[...] * m.astype(jnp.float32)


def _sc_body(vb_hbm, w_hbm, out_hbm, acc, vb0, w0, vb1, w1, sem0, sem1):
    fid = lax.axis_index("s") * 2 + lax.axis_index("c")
    bufsets = ((vb0, w0), (vb1, w1))
    sems = (sem0, sem1)

    @pl.when(fid < F)
    def _():
        zeros16 = jnp.zeros((16,), jnp.float32)

        def fire(ci):
            base = ci * CH
            bv, bw = bufsets[ci & 1]
            sem = sems[ci & 1]
            return [
                pltpu.async_copy(vb_hbm.at[pl.ds(base, CH)], bv, sem),
                pltpu.async_copy(w_hbm.at[fid, pl.ds(base, CH)], bw, sem),
            ]

        hs = fire(0)

        @plsc.parallel_loop(0, NVOX, step=16, unroll=8)
        def _zero(i):
            acc[pl.ds(i, 16)] = zeros16

        for ci in range(NCHUNK):
            hs_next = fire(ci + 1) if ci + 1 < NCHUNK else None
            for h in hs:
                h.wait()
            hs = hs_next
            bv, bw = bufsets[ci & 1]

            @plsc.parallel_loop(0, CH, step=16, unroll=4)
            def _group(off):
                vb = bv[pl.ds(off, 16)]
                w = bw[pl.ds(off, 16)]
                plsc.addupdate_scatter(acc, [vb], w, mask=vb < NVOX)

        pltpu.sync_copy(acc, out_hbm.at[fid])


def _conv_body(p_hbm, o_ref, buf, sem):
    f = pl.program_id(0)
    par = f % 2
    nxt = (f + 1) % 2

    @pl.when(f == 0)
    def _():
        pltpu.make_async_copy(p_hbm.at[0], buf.at[0], sem.at[0]).start()

    @pl.when(f + 1 < F)
    def _():
        pltpu.make_async_copy(p_hbm.at[f + 1], buf.at[nxt], sem.at[nxt]).start()

    pltpu.make_async_copy(p_hbm.at[f], buf.at[par], sem.at[par]).wait()

    zrow = jnp.zeros((1, D), jnp.float32)
    zcol = jnp.zeros((D, 1), jnp.float32)
    zm1, z0 = None, buf[par, 0]
    for z in range(D):
        zp1 = buf[par, z + 1] if z + 1 < D else None
        zs = z0
        if zm1 is not None:
            zs = zs + zm1
        if zp1 is not None:
            zs = zs + zp1
        zm1, z0 = z0, zp1
        t = (
            zs
            + jnp.concatenate([zs[1:, :], zrow], axis=0)
            + jnp.concatenate([zrow, zs[: D - 1, :]], axis=0)
        )
        o_ref[0, z] = (
            t
            + jnp.concatenate([t[:, 1:], zcol], axis=1)
            + jnp.concatenate([zcol, t[:, : D - 1]], axis=1)
        )


@jax.jit
def kernel(xyz, feat):
    xyzT = jnp.zeros((3, NPAD), jnp.float32).at[:, :N].set(xyz.T)
    featT = jnp.zeros((F, NPAD), jnp.float32).at[:, :N].set(feat.T)

    vb2, wT = pl.pallas_call(
        _bin_body,
        out_shape=[
            jax.ShapeDtypeStruct((1, NPAD), jnp.int32),
            jax.ShapeDtypeStruct((F, NPAD), jnp.float32),
        ],
    )(xyzT, featT)

    sc = pl.kernel(
        _sc_body,
        out_type=jax.ShapeDtypeStruct((F, NVOX), jnp.float32),
        mesh=plsc.VectorSubcoreMesh(core_axis_name="c", subcore_axis_name="s"),
        compiler_params=pltpu.CompilerParams(needs_layout_passes=False),
        scratch_types=[
            pltpu.VMEM((NVOX,), jnp.float32),
            pltpu.VMEM((CH,), jnp.int32),
            pltpu.VMEM((CH,), jnp.float32),
            pltpu.VMEM((CH,), jnp.int32),
            pltpu.VMEM((CH,), jnp.float32),
            pltpu.SemaphoreType.DMA,
            pltpu.SemaphoreType.DMA,
        ],
    )
    planes = sc(vb2.reshape(NPAD), wT)

    return pl.pallas_call(
        _conv_body,
        grid=(F,),
        in_specs=[pl.BlockSpec(memory_space=pl.ANY)],
        out_specs=pl.BlockSpec((1, D, D, D), lambda i: (i, 0, 0, 0)),
        out_shape=jax.ShapeDtypeStruct((F, D, D, D), jnp.float32),
        scratch_shapes=[
            pltpu.VMEM((2, D, D, D), jnp.float32),
            pltpu.SemaphoreType.DMA((2,)),
        ],
    )(planes.reshape(F, D, D, D))
